# Initial kernel scaffold; baseline (speedup 1.0000x reference)
import jax
import jax.numpy as jnp
from jax.experimental import pallas as pl


def kernel(tokens, table):
    raise NotImplementedError("write your pallas kernel here")


# speedup vs baseline: 5.7540x; 5.7540x over previous
import functools
import math

import jax
import jax.numpy as jnp
from jax import lax
from jax.experimental import pallas as pl
from jax.experimental.pallas import tpu as pltpu
from jax.experimental.pallas import tpu_sc as plsc

VOCAB = 100000
EMB = 128
B, L = 4096, 200
SCALE = math.sqrt(EMB)

NC, NS = 2, 16
NW = NC * NS
NTOK = B * L
N_PER_W = NTOK // NW
CH = 128
NCH = N_PER_W // CH

_SCALE_ROWS = 2000


def _scale_body(t_ref, o_ref):
    o_ref[...] = t_ref[...] * SCALE


def _scale_table(table):
    return pl.pallas_call(
        _scale_body,
        out_shape=jax.ShapeDtypeStruct((VOCAB, EMB), jnp.float32),
        grid=(VOCAB // _SCALE_ROWS,),
        in_specs=[pl.BlockSpec((_SCALE_ROWS, EMB), lambda i: (i, 0))],
        out_specs=pl.BlockSpec((_SCALE_ROWS, EMB), lambda i: (i, 0)),
    )(table)


def _gather_body(table_hbm, idx_hbm, out_hbm, idx_v, rows_v, gsem):
    wid = lax.axis_index("s") * NC + lax.axis_index("c")
    base = wid * N_PER_W

    pltpu.sync_copy(idx_hbm.at[wid], idx_v)

    def body(j, carry):
        pltpu.async_copy(table_hbm.at[idx_v.at[j]], rows_v, gsem).wait()
        pltpu.sync_copy(rows_v, out_hbm.at[pl.ds(base + j * CH, CH)])
        return carry

    lax.fori_loop(0, NCH, body, 0)


def _sc_gather(table, idx):
    mesh = plsc.VectorSubcoreMesh(core_axis_name="c", subcore_axis_name="s")
    run = functools.partial(
        pl.kernel,
        mesh=mesh,
        out_type=jax.ShapeDtypeStruct((NTOK, EMB), jnp.float32),
        scratch_types=[
            pltpu.VMEM((NCH, CH), jnp.int32),
            pltpu.VMEM((CH, EMB), jnp.float32),
            pltpu.SemaphoreType.DMA,
        ],
    )(_gather_body)
    return run(table, idx)


def kernel(tokens, table):
    idx = tokens.astype(jnp.int32).reshape(NW, NCH, CH)
    scaled = _scale_table(table)
    out = _sc_gather(scaled, idx)
    return out.reshape(B, L, EMB)


# speedup vs baseline: 7.9099x; 1.3747x over previous
import functools
import math

import jax
import jax.numpy as jnp
from jax import lax
from jax.experimental import pallas as pl
from jax.experimental.pallas import tpu as pltpu
from jax.experimental.pallas import tpu_sc as plsc

VOCAB = 100000
EMB = 128
B, L = 4096, 200
SCALE = math.sqrt(EMB)

NC, NS = 2, 16
NW = NC * NS
NTOK = B * L
N_PER_W = NTOK // NW
CH = 128
NCH = N_PER_W // CH

_SCALE_ROWS = 2000


def _scale_body(t_ref, o_ref):
    o_ref[...] = t_ref[...] * SCALE


def _scale_table(table):
    return pl.pallas_call(
        _scale_body,
        out_shape=jax.ShapeDtypeStruct((VOCAB, EMB), jnp.float32),
        grid=(VOCAB // _SCALE_ROWS,),
        in_specs=[pl.BlockSpec((_SCALE_ROWS, EMB), lambda i: (i, 0))],
        out_specs=pl.BlockSpec((_SCALE_ROWS, EMB), lambda i: (i, 0)),
    )(table)


NBUF = 5
NG = NCH // NBUF


def _gather_body(table_hbm, idx_hbm, out_hbm, idx_v, rows_v, *sems):
    gsem, ssem = sems[:NBUF], sems[NBUF:]
    wid = lax.axis_index("s") * NC + lax.axis_index("c")
    base = wid * N_PER_W

    pltpu.sync_copy(idx_hbm.at[wid], idx_v)

    def group(g, carry):
        j0 = g * NBUF
        descs = []
        for b in range(NBUF):
            @pl.when(g > 0)
            def _(b=b):
                pltpu.make_async_copy(
                    rows_v.at[b], out_hbm.at[pl.ds(base, CH)], ssem[b]
                ).wait()

            descs.append(
                pltpu.async_copy(
                    table_hbm.at[idx_v.at[j0 + b]], rows_v.at[b], gsem[b]
                )
            )
        for b in range(NBUF):
            descs[b].wait()
            pltpu.async_copy(
                rows_v.at[b],
                out_hbm.at[pl.ds(base + (j0 + b) * CH, CH)],
                ssem[b],
            )
        return carry

    lax.fori_loop(0, NG, group, 0)

    for b in range(NBUF):
        pltpu.make_async_copy(
            rows_v.at[b], out_hbm.at[pl.ds(base, CH)], ssem[b]
        ).wait()


def _sc_gather(table, idx):
    mesh = plsc.VectorSubcoreMesh(core_axis_name="c", subcore_axis_name="s")
    run = functools.partial(
        pl.kernel,
        mesh=mesh,
        out_type=jax.ShapeDtypeStruct((NTOK, EMB), jnp.float32),
        scratch_types=[
            pltpu.VMEM((NCH, CH), jnp.int32),
            pltpu.VMEM((NBUF, CH, EMB), jnp.float32),
        ]
        + [pltpu.SemaphoreType.DMA] * (2 * NBUF),
    )(_gather_body)
    return run(table, idx)


def kernel(tokens, table):
    idx = tokens.astype(jnp.int32).reshape(NW, NCH, CH)
    scaled = _scale_table(table)
    out = _sc_gather(scaled, idx)
    return out.reshape(B, L, EMB)


# speedup vs baseline: 9.1105x; 1.1518x over previous
import functools
import math

import jax
import jax.numpy as jnp
from jax import lax
from jax.experimental import pallas as pl
from jax.experimental.pallas import tpu as pltpu
from jax.experimental.pallas import tpu_sc as plsc

VOCAB = 100000
EMB = 128
B, L = 4096, 200
SCALE = math.sqrt(EMB)

NC, NS = 2, 16
NW = NC * NS
NTOK = B * L
N_PER_W = NTOK // NW
CH = 128
NCH = N_PER_W // CH

_SCALE_ROWS = 2000


def _scale_body(t_ref, o_ref):
    o_ref[...] = t_ref[...] * SCALE


def _scale_table(table):
    return pl.pallas_call(
        _scale_body,
        out_shape=jax.ShapeDtypeStruct((VOCAB, EMB), jnp.float32),
        grid=(VOCAB // _SCALE_ROWS,),
        in_specs=[pl.BlockSpec((_SCALE_ROWS, EMB), lambda i: (i, 0))],
        out_specs=pl.BlockSpec((_SCALE_ROWS, EMB), lambda i: (i, 0)),
    )(table)


NBUF = 5
NG = NCH // NBUF


def _gather_body(table_hbm, idx_hbm, out_hbm, idx_v, rows_v, *sems):
    gsem, ssem = sems[:NBUF], sems[NBUF:]
    wid = lax.axis_index("s") * NC + lax.axis_index("c")
    base = wid * N_PER_W

    pltpu.sync_copy(idx_hbm.at[wid], idx_v)

    def group(g, carry):
        j0 = g * NBUF
        descs = []
        for b in range(NBUF):
            @pl.when(g > 0)
            def _(b=b):
                pltpu.make_async_copy(
                    rows_v.at[b], out_hbm.at[pl.ds(base, CH)], ssem[b]
                ).wait()

            descs.append(
                pltpu.async_copy(
                    table_hbm.at[idx_v.at[j0 + b]], rows_v.at[b], gsem[b]
                )
            )
        for b in range(NBUF):
            descs[b].wait()

            def row_pair(r, carry, b=b):
                for rr in range(2):
                    for c in range(EMB // 16):
                        v = rows_v[b, 2 * r + rr, pl.ds(c * 16, 16)]
                        rows_v[b, 2 * r + rr, pl.ds(c * 16, 16)] = v * SCALE
                return carry

            lax.fori_loop(0, CH // 2, row_pair, 0)
            pltpu.async_copy(
                rows_v.at[b],
                out_hbm.at[pl.ds(base + (j0 + b) * CH, CH)],
                ssem[b],
            )
        return carry

    lax.fori_loop(0, NG, group, 0)

    for b in range(NBUF):
        pltpu.make_async_copy(
            rows_v.at[b], out_hbm.at[pl.ds(base, CH)], ssem[b]
        ).wait()


def _sc_gather(table, idx):
    mesh = plsc.VectorSubcoreMesh(core_axis_name="c", subcore_axis_name="s")
    run = functools.partial(
        pl.kernel,
        mesh=mesh,
        out_type=jax.ShapeDtypeStruct((NTOK, EMB), jnp.float32),
        scratch_types=[
            pltpu.VMEM((NCH, CH), jnp.int32),
            pltpu.VMEM((NBUF, CH, EMB), jnp.float32),
        ]
        + [pltpu.SemaphoreType.DMA] * (2 * NBUF),
    )(_gather_body)
    return run(table, idx)


def kernel(tokens, table):
    idx = tokens.astype(jnp.int32).reshape(NW, NCH, CH)
    out = _sc_gather(table, idx)
    return out.reshape(B, L, EMB)


# speedup vs baseline: 9.1282x; 1.0019x over previous
import functools
import math

import jax
import jax.numpy as jnp
from jax import lax
from jax.experimental import pallas as pl
from jax.experimental.pallas import tpu as pltpu
from jax.experimental.pallas import tpu_sc as plsc

VOCAB = 100000
EMB = 128
B, L = 4096, 200
SCALE = math.sqrt(EMB)

NC, NS = 2, 16
NW = NC * NS
NTOK = B * L
N_PER_W = NTOK // NW
CH = 128
HCH = CH // 2
NCH = N_PER_W // CH

NBUF = 6
NG = NCH // NBUF
NTAIL = NCH - NG * NBUF


def _scale_rows(rows_v, b):
    def row_pair(r, carry):
        for rr in range(2):
            for c in range(EMB // 16):
                v = rows_v[b, 2 * r + rr, pl.ds(c * 16, 16)]
                rows_v[b, 2 * r + rr, pl.ds(c * 16, 16)] = v * SCALE
        return carry

    lax.fori_loop(0, CH // 2, row_pair, 0)


def _gather_body(table_hbm, idx_hbm, out_hbm, idx_v, rows_v, *sems):
    gsem, ssem = sems[: 2 * NBUF], sems[2 * NBUF :]
    wid = lax.axis_index("s") * NC + lax.axis_index("c")
    base = wid * N_PER_W

    pltpu.sync_copy(idx_hbm.at[wid], idx_v)

    def fire_gathers(j, b):
        descs = []
        for h in range(2):
            descs.append(
                pltpu.async_copy(
                    table_hbm.at[idx_v.at[j, pl.ds(h * HCH, HCH)]],
                    rows_v.at[b, pl.ds(h * HCH, HCH)],
                    gsem[2 * b + h],
                )
            )
        return descs

    def fire_scatter(j, b):
        return pltpu.async_copy(
            rows_v.at[b], out_hbm.at[pl.ds(base + j * CH, CH)], ssem[b]
        )

    def drain_scatter(b):
        pltpu.make_async_copy(
            rows_v.at[b], out_hbm.at[pl.ds(base, CH)], ssem[b]
        ).wait()

    def group(g, carry):
        j0 = g * NBUF
        descs = []
        for b in range(NBUF):
            @pl.when(g > 0)
            def _(b=b):
                drain_scatter(b)

            descs.append(fire_gathers(j0 + b, b))
        for b in range(NBUF):
            for d in descs[b]:
                d.wait()
            _scale_rows(rows_v, b)
            fire_scatter(j0 + b, b)
        return carry

    lax.fori_loop(0, NG, group, 0)

    tdescs = []
    for t in range(NTAIL):
        drain_scatter(t)
        tdescs.append(fire_gathers(NG * NBUF + t, t))
    for t in range(NTAIL):
        for d in tdescs[t]:
            d.wait()
        _scale_rows(rows_v, t)
        fire_scatter(NG * NBUF + t, t)

    for t in range(NTAIL):
        drain_scatter(t)
    for b in range(NTAIL, NBUF):
        drain_scatter(b)


def _sc_gather(table, idx):
    mesh = plsc.VectorSubcoreMesh(core_axis_name="c", subcore_axis_name="s")
    run = functools.partial(
        pl.kernel,
        mesh=mesh,
        out_type=jax.ShapeDtypeStruct((NTOK, EMB), jnp.float32),
        scratch_types=[
            pltpu.VMEM((NCH, CH), jnp.int32),
            pltpu.VMEM((NBUF, CH, EMB), jnp.float32),
        ]
        + [pltpu.SemaphoreType.DMA] * (3 * NBUF),
    )(_gather_body)
    return run(table, idx)


def kernel(tokens, table):
    idx = tokens.astype(jnp.int32).reshape(NW, NCH, CH)
    out = _sc_gather(table, idx)
    return out.reshape(B, L, EMB)


# speedup vs baseline: 9.1658x; 1.0041x over previous
import functools
import math

import jax
import jax.numpy as jnp
from jax import lax
from jax.experimental import pallas as pl
from jax.experimental.pallas import tpu as pltpu
from jax.experimental.pallas import tpu_sc as plsc

VOCAB = 100000
EMB = 128
B, L = 4096, 200
SCALE = math.sqrt(EMB)

NC, NS = 2, 16
NW = NC * NS
NTOK = B * L
N_PER_W = NTOK // NW
CH = 128
HCH = CH // 2
NCH = N_PER_W // CH

NBUF = 6
NPAIR = NBUF // 2
NG = NCH // NBUF
NTAIL = NCH - NG * NBUF


def _scale_chunk(rows_v, b):
    def row_pair(r, carry):
        for rr in range(2):
            for c in range(EMB // 16):
                row = b * CH + 2 * r + rr
                v = rows_v[row, pl.ds(c * 16, 16)]
                rows_v[row, pl.ds(c * 16, 16)] = v * SCALE
        return carry

    lax.fori_loop(0, CH // 2, row_pair, 0)


def _gather_body(table_hbm, idx_hbm, out_hbm, idx_v, rows_v, *sems):
    gsem, ssem = sems[: 2 * NBUF], sems[2 * NBUF :]
    wid = lax.axis_index("s") * NC + lax.axis_index("c")
    base = wid * N_PER_W

    pltpu.sync_copy(idx_hbm.at[wid], idx_v)

    def fire_gathers(j, b):
        descs = []
        for h in range(2):
            descs.append(
                pltpu.async_copy(
                    table_hbm.at[idx_v.at[j, pl.ds(h * HCH, HCH)]],
                    rows_v.at[pl.ds(b * CH + h * HCH, HCH)],
                    gsem[2 * b + h],
                )
            )
        return descs

    def fire_pair_scatter(j, p):
        return pltpu.async_copy(
            rows_v.at[pl.ds(2 * p * CH, 2 * CH)],
            out_hbm.at[pl.ds(base + j * CH, 2 * CH)],
            ssem[p],
        )

    def drain_pair_scatter(p):
        pltpu.make_async_copy(
            rows_v.at[pl.ds(2 * p * CH, 2 * CH)],
            out_hbm.at[pl.ds(base, 2 * CH)],
            ssem[p],
        ).wait()

    def group(g, carry):
        j0 = g * NBUF
        descs = []
        for b in range(NBUF):
            if b % 2 == 0:
                p = b // 2

                @pl.when(g > 0)
                def _(p=p):
                    drain_pair_scatter(p)

            descs.append(fire_gathers(j0 + b, b))
        for b in range(NBUF):
            for d in descs[b]:
                d.wait()
            _scale_chunk(rows_v, b)
            if b % 2 == 1:
                fire_pair_scatter(j0 + b - 1, b // 2)
        return carry

    lax.fori_loop(0, NG, group, 0)

    drain_pair_scatter(0)
    tdescs = [fire_gathers(NG * NBUF + t, t) for t in range(NTAIL)]
    for t in range(NTAIL):
        for d in tdescs[t]:
            d.wait()
        _scale_chunk(rows_v, t)
    fire_pair_scatter(NG * NBUF, 0)

    for p in range(NPAIR):
        drain_pair_scatter(p)


def _sc_gather(table, idx):
    mesh = plsc.VectorSubcoreMesh(core_axis_name="c", subcore_axis_name="s")
    run = functools.partial(
        pl.kernel,
        mesh=mesh,
        out_type=jax.ShapeDtypeStruct((NTOK, EMB), jnp.float32),
        scratch_types=[
            pltpu.VMEM((NCH, CH), jnp.int32),
            pltpu.VMEM((NBUF * CH, EMB), jnp.float32),
        ]
        + [pltpu.SemaphoreType.DMA] * (2 * NBUF + NPAIR),
    )(_gather_body)
    return run(table, idx)


def kernel(tokens, table):
    idx = tokens.astype(jnp.int32).reshape(NW, NCH, CH)
    out = _sc_gather(table, idx)
    return out.reshape(B, L, EMB)
